# baseline (speedup 1.0000x reference)
import jax
import jax.numpy as jnp
from jax import lax
from jax.experimental import pallas as pl
from jax.experimental.pallas import tpu as pltpu
from jax.experimental.pallas import tpu_sc as plsc

_N = 10000
_E = 320000
_D = 128
_T = 4
_F = 32
_ED = 16
_L = 3

_NC, _NS = 2, 16
_NW = _NC * _NS
_NPT = 320
_NPC = _NPT * _NS
_NPAD = _NW * _NPT
_CAP = 16384
_CH = 128
_SCAN = 4000
_CNTROW = 336

_HP = lax.Precision.HIGHEST
_F32 = jnp.float32

_GDN = lax.GatherDimensionNumbers(offset_dims=(), collapsed_slice_dims=(0,),
                                  start_index_map=(0,))


def _gather16(v, idx):
    return lax.gather(v, idx[:, None], _GDN, (1,),
                      mode=lax.GatherScatterMode.PROMISE_IN_BOUNDS)


def _prefix16(x):
    iot = lax.iota(jnp.int32, 16)
    cs = x
    for k in (1, 2, 4, 8):
        sh = _gather16(cs, jnp.maximum(iot - k, 0))
        cs = cs + jnp.where(iot >= k, sh, 0)
    return cs


def _sc_prep_body(dst_h, src_h,
                  eidl_h, srcl_h, dstl_h, cnts_h, cntn_h,
                  dbuf, sbuf, leid, lsrc, ldst, cntb, cv):
    c = lax.axis_index("c")
    s = lax.axis_index("s")
    w = c * _NS + s
    lo = w * _NPT
    hi = lo + _NPT
    trash = c * _NPC + _NPC
    iot = lax.iota(jnp.int32, 16)
    z16i = jnp.zeros((16,), jnp.int32)
    tr16 = jnp.full((16,), trash, jnp.int32)
    z16f = jnp.zeros((16,), _F32)

    def _init(i, _):
        sl = pl.ds(i * 16, 16)
        leid[sl] = z16i
        lsrc[sl] = z16i
        ldst[sl] = tr16
        return 0
    lax.fori_loop(0, _CAP // 16, _init, 0)

    def _zc(i, _):
        cntb[pl.ds(i * 16, 16)] = z16f
        return 0
    lax.fori_loop(0, _CNTROW // 16, _zc, 0)

    def _blk(b, off):
        pltpu.sync_copy(dst_h.at[pl.ds(b * _SCAN, _SCAN)], dbuf)
        pltpu.sync_copy(src_h.at[pl.ds(b * _SCAN, _SCAN)], sbuf)

        def _ib(i, off):
            sl = pl.ds(i * 16, 16)
            d = dbuf[sl]
            sv = sbuf[sl]
            m = (d >= lo) & (d < hi)
            cs = plsc.cumsum(jnp.where(m, 1, 0))
            pos = jnp.where(m, off + cs - 1, _CAP - 16 + iot)
            eid = iot + (b * _SCAN + i * 16)
            plsc.store_scatter(leid, [pos], eid)
            plsc.store_scatter(lsrc, [pos], sv)
            plsc.store_scatter(ldst, [pos], d)
            return off + cs[15]
        return lax.fori_loop(0, _SCAN // 16, _ib, off)
    n_w = lax.fori_loop(0, _E // _SCAN, _blk, jnp.int32(0))

    ones16 = jnp.ones((16,), _F32)

    def _cb(g, _):
        d16 = ldst[pl.ds(g * 16, 16)]
        dl16 = jnp.clip(d16 - lo, 0, _CNTROW - 1)
        plsc.addupdate_scatter(cntb, [dl16], ones16)
        return 0
    lax.fori_loop(0, (n_w + 15) // 16, _cb, 0)

    pltpu.sync_copy(leid, eidl_h.at[pl.ds(w * _CAP, _CAP)])
    pltpu.sync_copy(lsrc, srcl_h.at[pl.ds(w * _CAP, _CAP)])
    pltpu.sync_copy(ldst, dstl_h.at[pl.ds(w * _CAP, _CAP)])
    cv[...] = jnp.full((16,), n_w, jnp.int32)
    pltpu.sync_copy(cv, cnts_h.at[pl.ds(w * 16, 16)])
    pltpu.sync_copy(cntb, cntn_h.at[pl.ds(w * _CNTROW, _CNTROW)])


def _sc_prep(dst, src):
    mesh = plsc.VectorSubcoreMesh(core_axis_name="c", subcore_axis_name="s",
                                  num_cores=_NC, num_subcores=_NS)
    f = pl.kernel(
        _sc_prep_body,
        compiler_params=pltpu.CompilerParams(needs_layout_passes=False),
        out_type=(
            jax.ShapeDtypeStruct((_NW * _CAP,), jnp.int32),
            jax.ShapeDtypeStruct((_NW * _CAP,), jnp.int32),
            jax.ShapeDtypeStruct((_NW * _CAP,), jnp.int32),
            jax.ShapeDtypeStruct((_NW * 16,), jnp.int32),
            jax.ShapeDtypeStruct((_NW * _CNTROW,), _F32),
        ),
        mesh=mesh,
        scratch_types=[
            pltpu.VMEM((_SCAN,), jnp.int32),
            pltpu.VMEM((_SCAN,), jnp.int32),
            pltpu.VMEM((_CAP,), jnp.int32),
            pltpu.VMEM((_CAP,), jnp.int32),
            pltpu.VMEM((_CAP,), jnp.int32),
            pltpu.VMEM((_CNTROW,), _F32),
            pltpu.VMEM((16,), jnp.int32),
        ],
    )
    return f(dst, src)


_CH2 = 64
_BLKE = 1024
_LSTG = _BLKE + 2 * _CH2


def _sc_layer_body(b_h, c_h, eidl_h, srcl_h, dstl_h, cnts_h,
                   *refs):
    ss_o = refs[0:8]
    mm_o = refs[8:16]
    (leb, lsb, ldb, dv0, dv1, bb0, bb1, cb0, cb1) = refs[16:25]
    sacc = refs[25:33]
    macc = refs[33:41]
    (cv, sb0, sb1, sc0, sc1) = refs[41:46]
    c = lax.axis_index("c")
    s = lax.axis_index("s")
    w = c * _NS + s
    lo = w * _NPT

    pltpu.sync_copy(cnts_h.at[pl.ds(w * 16, 16)], cv)
    n_w = cv[...][0]

    ninf = jnp.full((16,), -jnp.inf, _F32)
    z16f = jnp.zeros((16,), _F32)

    def _mi(r, _):
        for j in range(8):
            macc[j][pl.ds(r * 16, 16)] = ninf
            sacc[j][pl.ds(r * 16, 16)] = z16f
        return 0
    lax.fori_loop(0, _NPT + 1, _mi, 0)

    lbase = w * _CAP

    def _load_block(b):
        off = lbase + b * _BLKE
        pltpu.sync_copy(eidl_h.at[pl.ds(off, _LSTG)], leb)
        pltpu.sync_copy(srcl_h.at[pl.ds(off, _LSTG)], lsb)
        pltpu.sync_copy(dstl_h.at[pl.ds(off, _LSTG)], ldb)

    def _copy_dv(dv, off):
        def _cp(j, _):
            dv[pl.ds(j * 16, 16)] = ldb[pl.ds(off + j * 16, 16)]
            return 0
        lax.fori_loop(0, _CH2 // 16, _cp, 0)

    def _issue(off, bb, cb, sb, sc):
        db = pltpu.async_copy(b_h.at[lsb.at[pl.ds(off, _CH2)]], bb, sb)
        dc = pltpu.async_copy(c_h.at[leb.at[pl.ds(off, _CH2)]], cb, sc)
        return db, dc

    def _drain(bb, cb, sb, sc):
        pltpu.make_async_copy(b_h.at[pl.ds(0, _CH2)], bb, sb).wait()
        pltpu.make_async_copy(c_h.at[pl.ds(0, _CH2)], cb, sc).wait()

    def _compute(bb, cb, dv):
        def _eg(g, _):
            d16 = dv[pl.ds(g * 16, 16)]
            dl16 = jnp.clip(d16 - lo, 0, _NPT)
            for lane in range(16):
                dl = dl16[lane]
                e = g * 16 + lane
                dsl = pl.ds(dl * 16, 16)
                for j in range(8):
                    sl = pl.ds(j * 16, 16)
                    gv = bb[e, sl] + cb[e, sl]
                    plsc.addupdate(sacc[j].at[dsl], gv)
                    macc[j][dsl] = jnp.maximum(macc[j][dsl], gv)
            return 0
        lax.fori_loop(0, _CH2 // 16, _eg, 0)

    _load_block(0)
    _copy_dv(dv0, 0)
    _issue(0, bb0, cb0, sb0, sc0)

    nb = (n_w + _BLKE - 1) // _BLKE

    def _blk(b, _):
        _load_block(b)

        def _cpair(cp, _):
            o1 = (2 * cp + 1) * _CH2
            _copy_dv(dv1, o1)
            _issue(o1, bb1, cb1, sb1, sc1)
            _drain(bb0, cb0, sb0, sc0)
            _compute(bb0, cb0, dv0)
            _copy_dv(dv0, o1 + _CH2)
            _issue(o1 + _CH2, bb0, cb0, sb0, sc0)
            _drain(bb1, cb1, sb1, sc1)
            _compute(bb1, cb1, dv1)
            return 0
        lax.fori_loop(0, _BLKE // (2 * _CH2), _cpair, 0)
        return 0
    lax.fori_loop(0, nb, _blk, 0)

    _drain(bb0, cb0, sb0, sc0)

    for j in range(8):
        pltpu.sync_copy(macc[j].at[pl.ds(0, _NPT * 16)],
                        mm_o[j].at[pl.ds(lo * 16, _NPT * 16)])
        pltpu.sync_copy(sacc[j].at[pl.ds(0, _NPT * 16)],
                        ss_o[j].at[pl.ds(lo * 16, _NPT * 16)])


def _sc_layer(b, cmat, eidl, srcl, dstl, cnts):
    mesh = plsc.VectorSubcoreMesh(core_axis_name="c", subcore_axis_name="s",
                                  num_cores=_NC, num_subcores=_NS)
    f = pl.kernel(
        _sc_layer_body,
        compiler_params=pltpu.CompilerParams(needs_layout_passes=False),
        out_type=tuple(jax.ShapeDtypeStruct((_NPAD * 16,), _F32)
                       for _ in range(16)),
        mesh=mesh,
        scratch_types=(
            [pltpu.VMEM((_LSTG,), jnp.int32)] * 3
            + [pltpu.VMEM((_CH2,), jnp.int32)] * 2
            + [pltpu.VMEM((_CH2, _D), _F32)] * 4
            + [pltpu.VMEM(((_NPT + 1) * 16,), _F32)] * 16
            + [pltpu.VMEM((16,), jnp.int32)]
            + [pltpu.SemaphoreType.DMA] * 4
        ),
    )
    outs = f(b, cmat, eidl, srcl, dstl, cnts)
    cols = [o.reshape(_NPAD, 16) for o in outs]
    ss = jnp.concatenate(cols[0:8], axis=1)
    mm = jnp.concatenate(cols[8:16], axis=1)
    return ss, mm


def _dot(a, b):
    return lax.dot(a, b, precision=_HP, preferred_element_type=_F32)


def _mm_ab(x, wd, ws):
    def body(x_ref, wd_ref, ws_ref, a_ref, b_ref):
        xv = x_ref[...]
        a_ref[...] = _dot(xv, wd_ref[...])
        b_ref[...] = _dot(xv, ws_ref[...])

    return pl.pallas_call(
        body,
        grid=(10,),
        in_specs=[
            pl.BlockSpec((_N // 10, _D), lambda i: (i, 0)),
            pl.BlockSpec((_D, _D), lambda i: (0, 0)),
            pl.BlockSpec((_D, _D), lambda i: (0, 0)),
        ],
        out_specs=[pl.BlockSpec((_N // 10, _D), lambda i: (i, 0))] * 2,
        out_shape=[jax.ShapeDtypeStruct((_N, _D), _F32)] * 2,
    )(x, wd, ws)


def _mm_c(ea, me, ce8):
    blk = 8000

    def body(ea_ref, me_ref, ce_ref, o_ref):
        o_ref[...] = _dot(ea_ref[...], me_ref[...]) + ce_ref[0:1, :]

    return pl.pallas_call(
        body,
        grid=(_E // blk,),
        in_specs=[
            pl.BlockSpec((blk, _ED), lambda i: (i, 0)),
            pl.BlockSpec((_ED, _D), lambda i: (0, 0)),
            pl.BlockSpec((8, _D), lambda i: (0, 0)),
        ],
        out_specs=pl.BlockSpec((blk, _D), lambda i: (i, 0)),
        out_shape=jax.ShapeDtypeStruct((_E, _D), _F32),
    )(ea, me, ce8)


def _mm_post(x, a, ss, mm, cntb, pstack, cv8):
    blk = _N // 10

    def body(x_ref, a_ref, ss_ref, mm_ref, cnt_ref, p_ref, cv_ref, o_ref):
        xv = x_ref[...]
        av = a_ref[...]
        cnt = cnt_ref[...]
        sfull = cnt * av + ss_ref[...]
        mean = sfull / jnp.maximum(cnt, 1.0)
        mx = jnp.where(cnt > 0.0, av + mm_ref[...], 0.0)
        p = p_ref[...]
        o = (_dot(xv, p[0:_D]) + _dot(mean, p[_D:2 * _D])
             + _dot(sfull, p[2 * _D:3 * _D]) + _dot(mx, p[3 * _D:4 * _D])
             + cv_ref[0:1, :])
        o_ref[...] = jnp.maximum(o, 0.0)

    return pl.pallas_call(
        body,
        grid=(10,),
        in_specs=[
            pl.BlockSpec((blk, _D), lambda i: (i, 0)),
            pl.BlockSpec((blk, _D), lambda i: (i, 0)),
            pl.BlockSpec((blk, _D), lambda i: (i, 0)),
            pl.BlockSpec((blk, _D), lambda i: (i, 0)),
            pl.BlockSpec((blk, _D), lambda i: (i, 0)),
            pl.BlockSpec((4 * _D, _D), lambda i: (0, 0)),
            pl.BlockSpec((8, _D), lambda i: (0, 0)),
        ],
        out_specs=pl.BlockSpec((blk, _D), lambda i: (i, 0)),
        out_shape=jax.ShapeDtypeStruct((_N, _D), _F32),
    )(x, a, ss, mm, cntb, pstack, cv8)


def _block_diag(wl):
    out = jnp.zeros((_L, _D, _D), _F32)
    for t in range(_T):
        out = out.at[:, t * _F:(t + 1) * _F, t * _F:(t + 1) * _F].set(wl[:, t])
    return out


def kernel(x, edge_index, edge_attr, W_edge, b_edge, W_pre, b_pre, W_post,
           b_post, W_lin, b_lin):
    src = edge_index[0]
    dst = edge_index[1]

    bdd = _block_diag(W_pre[:, :, 0:_F, :])
    bds = _block_diag(W_pre[:, :, _F:2 * _F, :])
    wecat = jnp.concatenate([W_pre[:, t, 2 * _F:3 * _F, :]
                             for t in range(_T)], axis=-1)
    me = jnp.einsum("lef,lfd->led", W_edge, wecat, precision=_HP)
    ce = (jnp.einsum("lf,lfd->ld", b_edge, wecat, precision=_HP)
          + b_pre.reshape(_L, _D))
    pparts = []
    for p in range(4):
        bdp = _block_diag(W_post[:, :, p * _F:(p + 1) * _F, :])
        pparts.append(jnp.einsum("lab,lbc->lac", bdp, W_lin, precision=_HP))
    pstack = jnp.concatenate(pparts, axis=1)
    cvec = (jnp.einsum("ld,lde->le", b_post.reshape(_L, _D), W_lin,
                       precision=_HP) + b_lin)

    eidl, srcl, dstl, cnts, cntn = _sc_prep(dst, src)
    cnt_flat = cntn.reshape(_NW, _CNTROW)[:, :_NPT].reshape(_NPAD)[:_N]
    cntb = jnp.broadcast_to(cnt_flat[:, None], (_N, _D))

    h = x
    for l in range(_L):
        ce8 = jnp.tile(ce[l][None, :], (8, 1))
        cv8 = jnp.tile(cvec[l][None, :], (8, 1))
        a, b = _mm_ab(h, bdd[l], bds[l])
        cmat = _mm_c(edge_attr, me[l], ce8)
        ss, mmx = _sc_layer(b, cmat, eidl, srcl, dstl, cnts)
        h = _mm_post(h, a, ss[:_N], mmx[:_N], cntb, pstack[l], cv8)
    return h


# speedup vs baseline: 1.3093x; 1.3093x over previous
import jax
import jax.numpy as jnp
from jax import lax
from jax.experimental import pallas as pl
from jax.experimental.pallas import tpu as pltpu
from jax.experimental.pallas import tpu_sc as plsc

_N = 10000
_E = 320000
_D = 128
_T = 4
_F = 32
_ED = 16
_L = 3

_NC, _NS = 2, 16
_NW = _NC * _NS
_NPT = 320
_NPC = _NPT * _NS
_NPAD = _NW * _NPT
_CAP = 16384
_CH = 128
_SCAN = 4000
_CNTROW = 336

_HP = lax.Precision.HIGHEST
_F32 = jnp.float32

_GDN = lax.GatherDimensionNumbers(offset_dims=(), collapsed_slice_dims=(0,),
                                  start_index_map=(0,))


def _gather16(v, idx):
    return lax.gather(v, idx[:, None], _GDN, (1,),
                      mode=lax.GatherScatterMode.PROMISE_IN_BOUNDS)


def _prefix16(x):
    iot = lax.iota(jnp.int32, 16)
    cs = x
    for k in (1, 2, 4, 8):
        sh = _gather16(cs, jnp.maximum(iot - k, 0))
        cs = cs + jnp.where(iot >= k, sh, 0)
    return cs


def _sc_prep_body(dst_h, src_h,
                  eidl_h, srcl_h, dstl_h, cnts_h, cntn_h,
                  dbuf, sbuf, leid, lsrc, ldst, cntb, cv):
    c = lax.axis_index("c")
    s = lax.axis_index("s")
    w = c * _NS + s
    lo = w * _NPT
    hi = lo + _NPT
    trash = c * _NPC + _NPC
    iot = lax.iota(jnp.int32, 16)
    z16i = jnp.zeros((16,), jnp.int32)
    tr16 = jnp.full((16,), trash, jnp.int32)
    z16f = jnp.zeros((16,), _F32)

    def _init(i, _):
        sl = pl.ds(i * 16, 16)
        leid[sl] = z16i
        lsrc[sl] = z16i
        ldst[sl] = tr16
        return 0
    lax.fori_loop(0, _CAP // 16, _init, 0)

    def _zc(i, _):
        cntb[pl.ds(i * 16, 16)] = z16f
        return 0
    lax.fori_loop(0, _CNTROW // 16, _zc, 0)

    def _blk(b, off):
        pltpu.sync_copy(dst_h.at[pl.ds(b * _SCAN, _SCAN)], dbuf)
        pltpu.sync_copy(src_h.at[pl.ds(b * _SCAN, _SCAN)], sbuf)

        def _ib(i, off):
            sl = pl.ds(i * 16, 16)
            d = dbuf[sl]
            sv = sbuf[sl]
            m = (d >= lo) & (d < hi)
            cs = plsc.cumsum(jnp.where(m, 1, 0))
            pos = jnp.where(m, off + cs - 1, _CAP - 16 + iot)
            eid = iot + (b * _SCAN + i * 16)
            plsc.store_scatter(leid, [pos], eid)
            plsc.store_scatter(lsrc, [pos], sv)
            plsc.store_scatter(ldst, [pos], d)
            return off + cs[15]
        return lax.fori_loop(0, _SCAN // 16, _ib, off)
    n_w = lax.fori_loop(0, _E // _SCAN, _blk, jnp.int32(0))

    ones16 = jnp.ones((16,), _F32)

    def _cb(g, _):
        d16 = ldst[pl.ds(g * 16, 16)]
        dl16 = jnp.clip(d16 - lo, 0, _CNTROW - 1)
        plsc.addupdate_scatter(cntb, [dl16], ones16)
        return 0
    lax.fori_loop(0, (n_w + 15) // 16, _cb, 0)

    pltpu.sync_copy(leid, eidl_h.at[pl.ds(w * _CAP, _CAP)])
    pltpu.sync_copy(lsrc, srcl_h.at[pl.ds(w * _CAP, _CAP)])
    pltpu.sync_copy(ldst, dstl_h.at[pl.ds(w * _CAP, _CAP)])
    cv[...] = jnp.full((16,), n_w, jnp.int32)
    pltpu.sync_copy(cv, cnts_h.at[pl.ds(w * 16, 16)])
    pltpu.sync_copy(cntb, cntn_h.at[pl.ds(w * _CNTROW, _CNTROW)])


def _sc_prep(dst, src):
    mesh = plsc.VectorSubcoreMesh(core_axis_name="c", subcore_axis_name="s",
                                  num_cores=_NC, num_subcores=_NS)
    f = pl.kernel(
        _sc_prep_body,
        compiler_params=pltpu.CompilerParams(needs_layout_passes=False),
        out_type=(
            jax.ShapeDtypeStruct((_NW * _CAP,), jnp.int32),
            jax.ShapeDtypeStruct((_NW * _CAP,), jnp.int32),
            jax.ShapeDtypeStruct((_NW * _CAP,), jnp.int32),
            jax.ShapeDtypeStruct((_NW * 16,), jnp.int32),
            jax.ShapeDtypeStruct((_NW * _CNTROW,), _F32),
        ),
        mesh=mesh,
        scratch_types=[
            pltpu.VMEM((_SCAN,), jnp.int32),
            pltpu.VMEM((_SCAN,), jnp.int32),
            pltpu.VMEM((_CAP,), jnp.int32),
            pltpu.VMEM((_CAP,), jnp.int32),
            pltpu.VMEM((_CAP,), jnp.int32),
            pltpu.VMEM((_CNTROW,), _F32),
            pltpu.VMEM((16,), jnp.int32),
        ],
    )
    return f(dst, src)


_CH2 = 64
_BLKE = 1024
_LSTG = _BLKE + 2 * _CH2


def _sc_layer_body(b_h, c_h, eidl_h, srcl_h, dstl_h, cnts_h,
                   ss_h, mm_h,
                   leb, lsb, ldb, dv0, dv1, bb0, bb1, cb0, cb1,
                   maxacc, sumacc, cv, sb0, sb1, sc0, sc1):
    c = lax.axis_index("c")
    s = lax.axis_index("s")
    w = c * _NS + s
    lo = w * _NPT

    pltpu.sync_copy(cnts_h.at[pl.ds(w * 16, 16)], cv)
    n_w = cv[...][0]

    ninf = jnp.full((16,), -jnp.inf, _F32)
    z16f = jnp.zeros((16,), _F32)

    def _mi(r, _):
        for j in range(8):
            maxacc[r, pl.ds(j * 16, 16)] = ninf
        for j in range(8):
            sumacc[r, pl.ds(j * 16, 16)] = z16f
        return 0
    lax.fori_loop(0, _NPT + 1, _mi, 0)

    lbase = w * _CAP

    def _load_block(b):
        off = lbase + b * _BLKE
        pltpu.sync_copy(eidl_h.at[pl.ds(off, _LSTG)], leb)
        pltpu.sync_copy(srcl_h.at[pl.ds(off, _LSTG)], lsb)
        pltpu.sync_copy(dstl_h.at[pl.ds(off, _LSTG)], ldb)

    def _copy_dv(dv, off):
        def _cp(j, _):
            dv[pl.ds(j * 16, 16)] = ldb[pl.ds(off + j * 16, 16)]
            return 0
        lax.fori_loop(0, _CH2 // 16, _cp, 0)

    def _issue(off, bb, cb, sb, sc):
        db = pltpu.async_copy(b_h.at[lsb.at[pl.ds(off, _CH2)]], bb, sb)
        dc = pltpu.async_copy(c_h.at[leb.at[pl.ds(off, _CH2)]], cb, sc)
        return db, dc

    def _drain(bb, cb, sb, sc):
        pltpu.make_async_copy(b_h.at[pl.ds(0, _CH2)], bb, sb).wait()
        pltpu.make_async_copy(c_h.at[pl.ds(0, _CH2)], cb, sc).wait()

    def _compute(bb, cb, dv):
        def _eg(g, _):
            d16 = dv[pl.ds(g * 16, 16)]
            dl16 = jnp.clip(d16 - lo, 0, _NPT)
            for lane in range(16):
                dl = dl16[lane]
                e = g * 16 + lane
                sls = [pl.ds(j * 16, 16) for j in range(8)]
                gvs = [bb[e, sl] + cb[e, sl] for sl in sls]
                mxs = [jnp.maximum(maxacc[dl, sls[j]], gvs[j])
                       for j in range(8)]
                for j in range(8):
                    maxacc[dl, sls[j]] = mxs[j]
                for j in range(8):
                    plsc.addupdate(sumacc.at[dl, sls[j]], gvs[j])
            return 0
        lax.fori_loop(0, _CH2 // 16, _eg, 0)

    _load_block(0)
    _copy_dv(dv0, 0)
    _issue(0, bb0, cb0, sb0, sc0)

    nb = (n_w + _BLKE - 1) // _BLKE

    def _blk(b, _):
        _load_block(b)

        def _cpair(cp, _):
            o1 = (2 * cp + 1) * _CH2
            _copy_dv(dv1, o1)
            _issue(o1, bb1, cb1, sb1, sc1)
            _drain(bb0, cb0, sb0, sc0)
            _compute(bb0, cb0, dv0)
            _copy_dv(dv0, o1 + _CH2)
            _issue(o1 + _CH2, bb0, cb0, sb0, sc0)
            _drain(bb1, cb1, sb1, sc1)
            _compute(bb1, cb1, dv1)
            return 0
        lax.fori_loop(0, _BLKE // (2 * _CH2), _cpair, 0)
        return 0
    lax.fori_loop(0, nb, _blk, 0)

    _drain(bb0, cb0, sb0, sc0)

    pltpu.sync_copy(maxacc.at[pl.ds(0, _NPT)], mm_h.at[pl.ds(lo, _NPT)])
    pltpu.sync_copy(sumacc.at[pl.ds(0, _NPT)], ss_h.at[pl.ds(lo, _NPT)])


def _sc_layer(b, cmat, eidl, srcl, dstl, cnts):
    mesh = plsc.VectorSubcoreMesh(core_axis_name="c", subcore_axis_name="s",
                                  num_cores=_NC, num_subcores=_NS)
    f = pl.kernel(
        _sc_layer_body,
        compiler_params=pltpu.CompilerParams(needs_layout_passes=False),
        out_type=(
            jax.ShapeDtypeStruct((_NPAD, _D), _F32),
            jax.ShapeDtypeStruct((_NPAD, _D), _F32),
        ),
        mesh=mesh,
        scratch_types=(
            [pltpu.VMEM((_LSTG,), jnp.int32)] * 3
            + [pltpu.VMEM((_CH2,), jnp.int32)] * 2
            + [pltpu.VMEM((_CH2, _D), _F32)] * 4
            + [pltpu.VMEM((_NPT + 1, _D), _F32)] * 2
            + [pltpu.VMEM((16,), jnp.int32)]
            + [pltpu.SemaphoreType.DMA] * 4
        ),
    )
    return f(b, cmat, eidl, srcl, dstl, cnts)


def _dot(a, b):
    return lax.dot(a, b, precision=_HP, preferred_element_type=_F32)


def _mm_ab(x, wd, ws):
    def body(x_ref, wd_ref, ws_ref, a_ref, b_ref):
        xv = x_ref[...]
        a_ref[...] = _dot(xv, wd_ref[...])
        b_ref[...] = _dot(xv, ws_ref[...])

    return pl.pallas_call(
        body,
        grid=(10,),
        in_specs=[
            pl.BlockSpec((_N // 10, _D), lambda i: (i, 0)),
            pl.BlockSpec((_D, _D), lambda i: (0, 0)),
            pl.BlockSpec((_D, _D), lambda i: (0, 0)),
        ],
        out_specs=[pl.BlockSpec((_N // 10, _D), lambda i: (i, 0))] * 2,
        out_shape=[jax.ShapeDtypeStruct((_N, _D), _F32)] * 2,
    )(x, wd, ws)


def _mm_c(ea, me, ce8):
    blk = 8000

    def body(ea_ref, me_ref, ce_ref, o_ref):
        o_ref[...] = _dot(ea_ref[...], me_ref[...]) + ce_ref[0:1, :]

    return pl.pallas_call(
        body,
        grid=(_E // blk,),
        in_specs=[
            pl.BlockSpec((blk, _ED), lambda i: (i, 0)),
            pl.BlockSpec((_ED, _D), lambda i: (0, 0)),
            pl.BlockSpec((8, _D), lambda i: (0, 0)),
        ],
        out_specs=pl.BlockSpec((blk, _D), lambda i: (i, 0)),
        out_shape=jax.ShapeDtypeStruct((_E, _D), _F32),
    )(ea, me, ce8)


def _mm_post(x, a, ss, mm, cntb, pstack, cv8):
    blk = _N // 10

    def body(x_ref, a_ref, ss_ref, mm_ref, cnt_ref, p_ref, cv_ref, o_ref):
        xv = x_ref[...]
        av = a_ref[...]
        cnt = cnt_ref[...]
        sfull = cnt * av + ss_ref[...]
        mean = sfull / jnp.maximum(cnt, 1.0)
        mx = jnp.where(cnt > 0.0, av + mm_ref[...], 0.0)
        p = p_ref[...]
        o = (_dot(xv, p[0:_D]) + _dot(mean, p[_D:2 * _D])
             + _dot(sfull, p[2 * _D:3 * _D]) + _dot(mx, p[3 * _D:4 * _D])
             + cv_ref[0:1, :])
        o_ref[...] = jnp.maximum(o, 0.0)

    return pl.pallas_call(
        body,
        grid=(10,),
        in_specs=[
            pl.BlockSpec((blk, _D), lambda i: (i, 0)),
            pl.BlockSpec((blk, _D), lambda i: (i, 0)),
            pl.BlockSpec((blk, _D), lambda i: (i, 0)),
            pl.BlockSpec((blk, _D), lambda i: (i, 0)),
            pl.BlockSpec((blk, _D), lambda i: (i, 0)),
            pl.BlockSpec((4 * _D, _D), lambda i: (0, 0)),
            pl.BlockSpec((8, _D), lambda i: (0, 0)),
        ],
        out_specs=pl.BlockSpec((blk, _D), lambda i: (i, 0)),
        out_shape=jax.ShapeDtypeStruct((_N, _D), _F32),
    )(x, a, ss, mm, cntb, pstack, cv8)


def _block_diag(wl):
    out = jnp.zeros((_L, _D, _D), _F32)
    for t in range(_T):
        out = out.at[:, t * _F:(t + 1) * _F, t * _F:(t + 1) * _F].set(wl[:, t])
    return out


def kernel(x, edge_index, edge_attr, W_edge, b_edge, W_pre, b_pre, W_post,
           b_post, W_lin, b_lin):
    src = edge_index[0]
    dst = edge_index[1]

    bdd = _block_diag(W_pre[:, :, 0:_F, :])
    bds = _block_diag(W_pre[:, :, _F:2 * _F, :])
    wecat = jnp.concatenate([W_pre[:, t, 2 * _F:3 * _F, :]
                             for t in range(_T)], axis=-1)
    me = jnp.einsum("lef,lfd->led", W_edge, wecat, precision=_HP)
    ce = (jnp.einsum("lf,lfd->ld", b_edge, wecat, precision=_HP)
          + b_pre.reshape(_L, _D))
    pparts = []
    for p in range(4):
        bdp = _block_diag(W_post[:, :, p * _F:(p + 1) * _F, :])
        pparts.append(jnp.einsum("lab,lbc->lac", bdp, W_lin, precision=_HP))
    pstack = jnp.concatenate(pparts, axis=1)
    cvec = (jnp.einsum("ld,lde->le", b_post.reshape(_L, _D), W_lin,
                       precision=_HP) + b_lin)

    eidl, srcl, dstl, cnts, cntn = _sc_prep(dst, src)
    cnt_flat = cntn.reshape(_NW, _CNTROW)[:, :_NPT].reshape(_NPAD)[:_N]
    cntb = jnp.broadcast_to(cnt_flat[:, None], (_N, _D))

    h = x
    for l in range(_L):
        ce8 = jnp.tile(ce[l][None, :], (8, 1))
        cv8 = jnp.tile(cvec[l][None, :], (8, 1))
        a, b = _mm_ab(h, bdd[l], bds[l])
        cmat = _mm_c(edge_attr, me[l], ce8)
        ss, mmx = _sc_layer(b, cmat, eidl, srcl, dstl, cnts)
        h = _mm_post(h, a, ss[:_N], mmx[:_N], cntb, pstack[l], cv8)
    return h
